# Initial kernel scaffold; baseline (speedup 1.0000x reference)
import jax
import jax.numpy as jnp
from jax.experimental import pallas as pl


def kernel(positions, hidden_states, residual, input_ln_w, post_ln_w, W_qkv_a, q_a_ln_w, kv_a_ln_w, W_q_b, W_idx_k, idx_k_norm_w, idx_k_norm_b, W_idx_wts, W_idx_q_b, W_kv_b, W_o, W_gate, W_up, W_down):
    raise NotImplementedError("write your pallas kernel here")


# speedup vs baseline: 1.4729x; 1.4729x over previous
import functools

import jax
import jax.numpy as jnp
import numpy as np
from jax.experimental import pallas as pl
from jax.experimental.pallas import tpu as pltpu

T = 2048
D = 2048
H = 16
QL = 1536
KVL = 512
RD = 64
ND = 128
VD = 128
IH = 16
ID = 128
TOPK = 512
FF = 5632
EPS = 1e-6
NEG = -1e30

BT = 256
NBT = T // BT
BF = 512
NBF = FF // BF


def _rope_cs(pos_col):
    j = jax.lax.broadcasted_iota(jnp.int32, (1, RD // 2), 1).astype(jnp.float32)
    inv = jnp.exp(j * (-np.log(10000.0) / (RD // 2)))
    f = pos_col * inv
    return jnp.cos(f), jnp.sin(f)


def _rope(x, cos, sin):
    x1 = x[:, : RD // 2]
    x2 = x[:, RD // 2:]
    return jnp.concatenate([x1 * cos - x2 * sin, x2 * cos + x1 * sin], axis=1)


def _rms(x, w):
    var = jnp.mean(x * x, axis=-1, keepdims=True)
    return x * jax.lax.rsqrt(var + EPS) * w


def _k1_body(pos_ref, hs_ref, rs_ref, ilw_ref, wqkv_ref, qalw_ref, kvalw_ref,
             widxk_ref, iknw_ref, iknb_ref, widxw_ref,
             res_ref, qc_ref, kvc_ref, kpe_ref, ika_ref, ikb_ref, wts_ref):
    h0 = hs_ref[...] + rs_ref[...]
    res_ref[...] = h0
    h = _rms(h0, ilw_ref[...])
    qkv = jnp.dot(h, wqkv_ref[...], preferred_element_type=jnp.float32)
    cos, sin = _rope_cs(pos_ref[...])
    qc_ref[...] = _rms(qkv[:, :QL], qalw_ref[...])
    kvc_ref[...] = _rms(qkv[:, QL:QL + KVL], kvalw_ref[...])
    kpe_ref[...] = _rope(qkv[:, QL + KVL:], cos, sin)
    ik = jnp.dot(h, widxk_ref[...], preferred_element_type=jnp.float32)
    m = jnp.mean(ik, axis=-1, keepdims=True)
    v = jnp.mean((ik - m) ** 2, axis=-1, keepdims=True)
    ik = (ik - m) * jax.lax.rsqrt(v + EPS) * iknw_ref[...] + iknb_ref[...]
    ika_ref[...] = ik[:, :ID - RD]
    ikb_ref[...] = _rope(ik[:, ID - RD:], cos, sin)
    wts_ref[...] = jnp.dot(h, widxw_ref[...],
                           preferred_element_type=jnp.float32) * (IH ** -0.5)


def _k1(pos_col, hidden, resid, ilw, wqkv, qalw, kvalw, widxk, iknw, iknb, widxw):
    row = lambda i: (i, 0)
    fixed = lambda i: (0, 0)
    return pl.pallas_call(
        _k1_body,
        grid=(NBT,),
        in_specs=[
            pl.BlockSpec((BT, 1), row),
            pl.BlockSpec((BT, D), row),
            pl.BlockSpec((BT, D), row),
            pl.BlockSpec((1, D), fixed),
            pl.BlockSpec((D, QL + KVL + RD), fixed),
            pl.BlockSpec((1, QL), fixed),
            pl.BlockSpec((1, KVL), fixed),
            pl.BlockSpec((D, ID), fixed),
            pl.BlockSpec((1, ID), fixed),
            pl.BlockSpec((1, ID), fixed),
            pl.BlockSpec((D, IH), fixed),
        ],
        out_specs=[
            pl.BlockSpec((BT, D), row),
            pl.BlockSpec((BT, QL), row),
            pl.BlockSpec((BT, KVL), row),
            pl.BlockSpec((BT, RD), row),
            pl.BlockSpec((BT, ID - RD), row),
            pl.BlockSpec((BT, RD), row),
            pl.BlockSpec((BT, IH), row),
        ],
        out_shape=[
            jax.ShapeDtypeStruct((T, D), jnp.float32),
            jax.ShapeDtypeStruct((T, QL), jnp.float32),
            jax.ShapeDtypeStruct((T, KVL), jnp.float32),
            jax.ShapeDtypeStruct((T, RD), jnp.float32),
            jax.ShapeDtypeStruct((T, ID - RD), jnp.float32),
            jax.ShapeDtypeStruct((T, RD), jnp.float32),
            jax.ShapeDtypeStruct((T, IH), jnp.float32),
        ],
    )(pos_col, hidden, resid, ilw.reshape(1, D), wqkv, qalw.reshape(1, QL),
      kvalw.reshape(1, KVL), widxk, iknw.reshape(1, ID), iknb.reshape(1, ID),
      widxw)


def _k2a_body(pos_ref, qc_ref, wqn_ref, wqr_ref, qn_ref, qr_ref):
    qc = qc_ref[...]
    cos, sin = _rope_cs(pos_ref[...])
    for h in range(H):
        qn_ref[h] = jnp.dot(qc, wqn_ref[h], preferred_element_type=jnp.float32)
        qr = jnp.dot(qc, wqr_ref[h], preferred_element_type=jnp.float32)
        qr_ref[h] = _rope(qr, cos, sin)


def _k2a(pos_col, qc, wqn, wqr):
    return pl.pallas_call(
        _k2a_body,
        grid=(NBT,),
        in_specs=[
            pl.BlockSpec((BT, 1), lambda i: (i, 0)),
            pl.BlockSpec((BT, QL), lambda i: (i, 0)),
            pl.BlockSpec((H, QL, ND), lambda i: (0, 0, 0)),
            pl.BlockSpec((H, QL, RD), lambda i: (0, 0, 0)),
        ],
        out_specs=[
            pl.BlockSpec((H, BT, ND), lambda i: (0, i, 0)),
            pl.BlockSpec((H, BT, RD), lambda i: (0, i, 0)),
        ],
        out_shape=[
            jax.ShapeDtypeStruct((H, T, ND), jnp.float32),
            jax.ShapeDtypeStruct((H, T, RD), jnp.float32),
        ],
    )(pos_col, qc, wqn, wqr)


def _k2i_body(pos_ref, qc_ref, wia_ref, wib_ref, iqa_ref, iqb_ref):
    qc = qc_ref[...]
    cos, sin = _rope_cs(pos_ref[...])
    for h in range(IH):
        iqa_ref[h] = jnp.dot(qc, wia_ref[h], preferred_element_type=jnp.float32)
        iqb = jnp.dot(qc, wib_ref[h], preferred_element_type=jnp.float32)
        iqb_ref[h] = _rope(iqb, cos, sin)


def _k2i(pos_col, qc, wia, wib):
    return pl.pallas_call(
        _k2i_body,
        grid=(NBT,),
        in_specs=[
            pl.BlockSpec((BT, 1), lambda i: (i, 0)),
            pl.BlockSpec((BT, QL), lambda i: (i, 0)),
            pl.BlockSpec((IH, QL, ID - RD), lambda i: (0, 0, 0)),
            pl.BlockSpec((IH, QL, RD), lambda i: (0, 0, 0)),
        ],
        out_specs=[
            pl.BlockSpec((IH, BT, ID - RD), lambda i: (0, i, 0)),
            pl.BlockSpec((IH, BT, RD), lambda i: (0, i, 0)),
        ],
        out_shape=[
            jax.ShapeDtypeStruct((IH, T, ID - RD), jnp.float32),
            jax.ShapeDtypeStruct((IH, T, RD), jnp.float32),
        ],
    )(pos_col, qc, wia, wib)


def _k2b_body(kvc_ref, wkn_ref, wv_ref, kn_ref, v_ref):
    kvc = kvc_ref[...]
    for h in range(H):
        kn_ref[h] = jnp.dot(kvc, wkn_ref[h], preferred_element_type=jnp.float32)
        v_ref[h] = jnp.dot(kvc, wv_ref[h], preferred_element_type=jnp.float32)


def _k2b(kvc, wkn, wv):
    return pl.pallas_call(
        _k2b_body,
        grid=(NBT,),
        in_specs=[
            pl.BlockSpec((BT, KVL), lambda i: (i, 0)),
            pl.BlockSpec((H, KVL, ND), lambda i: (0, 0, 0)),
            pl.BlockSpec((H, KVL, VD), lambda i: (0, 0, 0)),
        ],
        out_specs=[
            pl.BlockSpec((H, BT, ND), lambda i: (0, i, 0)),
            pl.BlockSpec((H, BT, VD), lambda i: (0, i, 0)),
        ],
        out_shape=[
            jax.ShapeDtypeStruct((H, T, ND), jnp.float32),
            jax.ShapeDtypeStruct((H, T, VD), jnp.float32),
        ],
    )(kvc, wkn, wv)


def _k3_body(iqa_ref, iqb_ref, ika_ref, ikb_ref, wts_ref, sc_ref, th_ref):
    i = pl.program_id(0)
    acc = jnp.zeros((BT, T), jnp.float32)
    ika = ika_ref[...]
    ikb = ikb_ref[...]
    for h in range(IH):
        lg = jax.lax.dot_general(iqa_ref[h], ika,
                                 (((1,), (1,)), ((), ())),
                                 preferred_element_type=jnp.float32)
        lg = lg + jax.lax.dot_general(iqb_ref[h], ikb,
                                      (((1,), (1,)), ((), ())),
                                      preferred_element_type=jnp.float32)
        w = wts_ref[...][:, h:h + 1]
        acc = acc + jnp.maximum(lg, 0.0) * w
    acc = acc * (ID ** -0.5)
    rows = i * BT + jax.lax.broadcasted_iota(jnp.int32, (BT, T), 0)
    cols = jax.lax.broadcasted_iota(jnp.int32, (BT, T), 1)
    sc = jnp.where(cols <= rows, acc, NEG)
    sc_ref[...] = sc
    b = jax.lax.bitcast_convert_type(sc, jnp.uint32)
    sign = jnp.uint32(0x80000000)
    keys = jnp.where(b >= sign, ~b, b | sign)
    lo = jnp.zeros((BT, 1), jnp.uint32)
    for bit in range(31, -1, -1):
        cand = lo | jnp.uint32(1 << bit)
        cnt = jnp.sum((keys >= cand).astype(jnp.int32), axis=1, keepdims=True)
        lo = jnp.where(cnt >= TOPK, cand, lo)
    tb = jnp.where(lo >= sign, lo ^ sign, ~lo)
    th_ref[...] = jax.lax.bitcast_convert_type(tb, jnp.float32)


def _k3(iqa, iqb, ika, ikb, wts):
    return pl.pallas_call(
        _k3_body,
        grid=(NBT,),
        in_specs=[
            pl.BlockSpec((IH, BT, ID - RD), lambda i: (0, i, 0)),
            pl.BlockSpec((IH, BT, RD), lambda i: (0, i, 0)),
            pl.BlockSpec((T, ID - RD), lambda i: (0, 0)),
            pl.BlockSpec((T, RD), lambda i: (0, 0)),
            pl.BlockSpec((BT, IH), lambda i: (i, 0)),
        ],
        out_specs=[
            pl.BlockSpec((BT, T), lambda i: (i, 0)),
            pl.BlockSpec((BT, 1), lambda i: (i, 0)),
        ],
        out_shape=[
            jax.ShapeDtypeStruct((T, T), jnp.float32),
            jax.ShapeDtypeStruct((T, 1), jnp.float32),
        ],
    )(iqa, iqb, ika, ikb, wts)


def _k5_body(qn_ref, qr_ref, kn_ref, kpe_ref, v_ref, sc_ref, th_ref, o_ref):
    i = pl.program_id(0)
    scale = (ND + RD) ** -0.5
    lg = jax.lax.dot_general(qn_ref[0], kn_ref[0], (((1,), (1,)), ((), ())),
                             preferred_element_type=jnp.float32)
    lg = lg + jax.lax.dot_general(qr_ref[0], kpe_ref[...],
                                  (((1,), (1,)), ((), ())),
                                  preferred_element_type=jnp.float32)
    lg = lg * scale
    rows = i * BT + jax.lax.broadcasted_iota(jnp.int32, (BT, T), 0)
    cols = jax.lax.broadcasted_iota(jnp.int32, (BT, T), 1)
    mask = (sc_ref[...] >= th_ref[...]) & (cols <= rows)
    lg = jnp.where(mask, lg, NEG)
    m = jnp.max(lg, axis=1, keepdims=True)
    e = jnp.exp(lg - m)
    p = e / jnp.sum(e, axis=1, keepdims=True)
    o_ref[0] = jnp.dot(p, v_ref[0], preferred_element_type=jnp.float32)


def _k5(qn, qr, kn, kpe, v, sc, th):
    return pl.pallas_call(
        _k5_body,
        grid=(NBT, H),
        in_specs=[
            pl.BlockSpec((1, BT, ND), lambda i, h: (h, i, 0)),
            pl.BlockSpec((1, BT, RD), lambda i, h: (h, i, 0)),
            pl.BlockSpec((1, T, ND), lambda i, h: (h, 0, 0)),
            pl.BlockSpec((T, RD), lambda i, h: (0, 0)),
            pl.BlockSpec((1, T, VD), lambda i, h: (h, 0, 0)),
            pl.BlockSpec((BT, T), lambda i, h: (i, 0)),
            pl.BlockSpec((BT, 1), lambda i, h: (i, 0)),
        ],
        out_specs=pl.BlockSpec((1, BT, VD), lambda i, h: (h, i, 0)),
        out_shape=jax.ShapeDtypeStruct((H, T, VD), jnp.float32),
    )(qn, qr, kn, kpe, v, sc, th)


def _k6_body(ao_ref, wo_ref, res_ref, plw_ref, res2_ref, h2_ref):
    acc = jnp.zeros((BT, D), jnp.float32)
    for h in range(H):
        acc = acc + jnp.dot(ao_ref[h], wo_ref[h],
                            preferred_element_type=jnp.float32)
    res2 = acc + res_ref[...]
    res2_ref[...] = res2
    h2_ref[...] = _rms(res2, plw_ref[...])


def _k6(ao, wo_r, res, plw):
    return pl.pallas_call(
        _k6_body,
        grid=(NBT,),
        in_specs=[
            pl.BlockSpec((H, BT, VD), lambda i: (0, i, 0)),
            pl.BlockSpec((H, VD, D), lambda i: (0, 0, 0)),
            pl.BlockSpec((BT, D), lambda i: (i, 0)),
            pl.BlockSpec((1, D), lambda i: (0, 0)),
        ],
        out_specs=[
            pl.BlockSpec((BT, D), lambda i: (i, 0)),
            pl.BlockSpec((BT, D), lambda i: (i, 0)),
        ],
        out_shape=[
            jax.ShapeDtypeStruct((T, D), jnp.float32),
            jax.ShapeDtypeStruct((T, D), jnp.float32),
        ],
    )(ao, wo_r, res, plw.reshape(1, D))


def _k7_body(h2_ref, wg_ref, wu_ref, wd_ref, o_ref):
    @pl.when(pl.program_id(1) == 0)
    def _():
        o_ref[...] = jnp.zeros_like(o_ref)

    h2 = h2_ref[...]
    g = jnp.dot(h2, wg_ref[...], preferred_element_type=jnp.float32)
    u = jnp.dot(h2, wu_ref[...], preferred_element_type=jnp.float32)
    a = g * jax.lax.logistic(g) * u
    o_ref[...] += jnp.dot(a, wd_ref[...], preferred_element_type=jnp.float32)


def _k7(h2, wg, wu, wd):
    return pl.pallas_call(
        _k7_body,
        grid=(NBT, NBF),
        in_specs=[
            pl.BlockSpec((BT, D), lambda i, j: (i, 0)),
            pl.BlockSpec((D, BF), lambda i, j: (0, j)),
            pl.BlockSpec((D, BF), lambda i, j: (0, j)),
            pl.BlockSpec((BF, D), lambda i, j: (j, 0)),
        ],
        out_specs=pl.BlockSpec((BT, D), lambda i, j: (i, 0)),
        out_shape=jax.ShapeDtypeStruct((T, D), jnp.float32),
        compiler_params=pltpu.CompilerParams(
            dimension_semantics=("arbitrary", "arbitrary")),
    )(h2, wg, wu, wd)


def kernel(positions, hidden_states, residual, input_ln_w, post_ln_w, W_qkv_a,
           q_a_ln_w, kv_a_ln_w, W_q_b, W_idx_k, idx_k_norm_w, idx_k_norm_b,
           W_idx_wts, W_idx_q_b, W_kv_b, W_o, W_gate, W_up, W_down):
    pos_col = positions.astype(jnp.float32).reshape(T, 1)
    wq = W_q_b.reshape(QL, H, ND + RD)
    wqn = jnp.transpose(wq[:, :, :ND], (1, 0, 2))
    wqr = jnp.transpose(wq[:, :, ND:], (1, 0, 2))
    wi = W_idx_q_b.reshape(QL, IH, ID)
    wia = jnp.transpose(wi[:, :, :ID - RD], (1, 0, 2))
    wib = jnp.transpose(wi[:, :, ID - RD:], (1, 0, 2))
    wkv = W_kv_b.reshape(KVL, H, ND + VD)
    wkn = jnp.transpose(wkv[:, :, :ND], (1, 0, 2))
    wv = jnp.transpose(wkv[:, :, ND:], (1, 0, 2))
    wo_r = W_o.reshape(H, VD, D)

    res, qc, kvc, kpe, ika, ikb, wts = _k1(
        pos_col, hidden_states, residual, input_ln_w, W_qkv_a, q_a_ln_w,
        kv_a_ln_w, W_idx_k, idx_k_norm_w, idx_k_norm_b, W_idx_wts)
    qn, qr = _k2a(pos_col, qc, wqn, wqr)
    iqa, iqb = _k2i(pos_col, qc, wia, wib)
    kn, v = _k2b(kvc, wkn, wv)
    sc, th = _k3(iqa, iqb, ika, ikb, wts)
    ao = _k5(qn, qr, kn, kpe, v, sc, th)
    res2, h2 = _k6(ao, wo_r, res, post_ln_w)
    mlp_out = _k7(h2, W_gate, W_up, W_down)
    return (mlp_out, res2)


# speedup vs baseline: 1.6955x; 1.1511x over previous
import functools

import jax
import jax.numpy as jnp
import numpy as np
from jax.experimental import pallas as pl
from jax.experimental.pallas import tpu as pltpu

T = 2048
D = 2048
H = 16
QL = 1536
KVL = 512
RD = 64
ND = 128
VD = 128
IH = 16
ID = 128
TOPK = 512
FF = 5632
EPS = 1e-6
NEG = -1e30

BT = 256
NBT = T // BT
BF = 1408
NBF = FF // BF


def _rope_cs(pos_col):
    j = jax.lax.broadcasted_iota(jnp.int32, (1, RD // 2), 1).astype(jnp.float32)
    inv = jnp.exp(j * (-np.log(10000.0) / (RD // 2)))
    f = pos_col * inv
    return jnp.cos(f), jnp.sin(f)


def _rope(x, cos, sin):
    x1 = x[:, : RD // 2]
    x2 = x[:, RD // 2:]
    return jnp.concatenate([x1 * cos - x2 * sin, x2 * cos + x1 * sin], axis=1)


def _rms(x, w):
    var = jnp.mean(x * x, axis=-1, keepdims=True)
    return x * jax.lax.rsqrt(var + EPS) * w


def _k1_body(pos_ref, hs_ref, rs_ref, ilw_ref, wqkv_ref, qalw_ref, kvalw_ref,
             widxk_ref, iknw_ref, iknb_ref, widxw_ref,
             res_ref, qc_ref, kvc_ref, kpe_ref, ika_ref, ikb_ref, wts_ref):
    h0 = hs_ref[...] + rs_ref[...]
    res_ref[...] = h0
    h = _rms(h0, ilw_ref[...])
    qkv = jnp.dot(h, wqkv_ref[...], preferred_element_type=jnp.float32)
    cos, sin = _rope_cs(pos_ref[...])
    qc_ref[...] = _rms(qkv[:, :QL], qalw_ref[...])
    kvc_ref[...] = _rms(qkv[:, QL:QL + KVL], kvalw_ref[...])
    kpe_ref[...] = _rope(qkv[:, QL + KVL:], cos, sin).astype(jnp.bfloat16)
    ik = jnp.dot(h, widxk_ref[...], preferred_element_type=jnp.float32)
    m = jnp.mean(ik, axis=-1, keepdims=True)
    v = jnp.mean((ik - m) ** 2, axis=-1, keepdims=True)
    ik = (ik - m) * jax.lax.rsqrt(v + EPS) * iknw_ref[...] + iknb_ref[...]
    ika_ref[...] = ik[:, :ID - RD]
    ikb_ref[...] = _rope(ik[:, ID - RD:], cos, sin)
    wts_ref[...] = jnp.dot(h, widxw_ref[...],
                           preferred_element_type=jnp.float32) * (IH ** -0.5)


def _k1(pos_col, hidden, resid, ilw, wqkv, qalw, kvalw, widxk, iknw, iknb, widxw):
    row = lambda i: (i, 0)
    fixed = lambda i: (0, 0)
    return pl.pallas_call(
        _k1_body,
        grid=(NBT,),
        in_specs=[
            pl.BlockSpec((BT, 1), row),
            pl.BlockSpec((BT, D), row),
            pl.BlockSpec((BT, D), row),
            pl.BlockSpec((1, D), fixed),
            pl.BlockSpec((D, QL + KVL + RD), fixed),
            pl.BlockSpec((1, QL), fixed),
            pl.BlockSpec((1, KVL), fixed),
            pl.BlockSpec((D, ID), fixed),
            pl.BlockSpec((1, ID), fixed),
            pl.BlockSpec((1, ID), fixed),
            pl.BlockSpec((D, IH), fixed),
        ],
        out_specs=[
            pl.BlockSpec((BT, D), row),
            pl.BlockSpec((BT, QL), row),
            pl.BlockSpec((BT, KVL), row),
            pl.BlockSpec((BT, RD), row),
            pl.BlockSpec((BT, ID - RD), row),
            pl.BlockSpec((BT, RD), row),
            pl.BlockSpec((BT, IH), row),
        ],
        out_shape=[
            jax.ShapeDtypeStruct((T, D), jnp.float32),
            jax.ShapeDtypeStruct((T, QL), jnp.float32),
            jax.ShapeDtypeStruct((T, KVL), jnp.float32),
            jax.ShapeDtypeStruct((T, RD), jnp.bfloat16),
            jax.ShapeDtypeStruct((T, ID - RD), jnp.float32),
            jax.ShapeDtypeStruct((T, RD), jnp.float32),
            jax.ShapeDtypeStruct((T, IH), jnp.float32),
        ],
    )(pos_col, hidden, resid, ilw.reshape(1, D), wqkv, qalw.reshape(1, QL),
      kvalw.reshape(1, KVL), widxk, iknw.reshape(1, ID), iknb.reshape(1, ID),
      widxw)


def _k2a_body(pos_ref, qc_ref, wqn_ref, wqr_ref, qn_ref, qr_ref):
    qc = qc_ref[...].astype(jnp.bfloat16)
    cos, sin = _rope_cs(pos_ref[...])
    for h in range(H):
        qn = jnp.dot(qc, wqn_ref[h], preferred_element_type=jnp.float32)
        qn_ref[h] = qn.astype(jnp.bfloat16)
        qr = jnp.dot(qc, wqr_ref[h], preferred_element_type=jnp.float32)
        qr_ref[h] = _rope(qr, cos, sin).astype(jnp.bfloat16)


def _k2a(pos_col, qc, wqn, wqr):
    return pl.pallas_call(
        _k2a_body,
        grid=(NBT,),
        in_specs=[
            pl.BlockSpec((BT, 1), lambda i: (i, 0)),
            pl.BlockSpec((BT, QL), lambda i: (i, 0)),
            pl.BlockSpec((H, QL, ND), lambda i: (0, 0, 0)),
            pl.BlockSpec((H, QL, RD), lambda i: (0, 0, 0)),
        ],
        out_specs=[
            pl.BlockSpec((H, BT, ND), lambda i: (0, i, 0)),
            pl.BlockSpec((H, BT, RD), lambda i: (0, i, 0)),
        ],
        out_shape=[
            jax.ShapeDtypeStruct((H, T, ND), jnp.bfloat16),
            jax.ShapeDtypeStruct((H, T, RD), jnp.bfloat16),
        ],
    )(pos_col, qc, wqn, wqr)


def _k2i_body(pos_ref, qc_ref, wia_ref, wib_ref, iqa_ref, iqb_ref):
    qc = qc_ref[...]
    cos, sin = _rope_cs(pos_ref[...])
    for h in range(IH):
        iqa_ref[h] = jnp.dot(qc, wia_ref[h], preferred_element_type=jnp.float32)
        iqb = jnp.dot(qc, wib_ref[h], preferred_element_type=jnp.float32)
        iqb_ref[h] = _rope(iqb, cos, sin)


def _k2i(pos_col, qc, wia, wib):
    return pl.pallas_call(
        _k2i_body,
        grid=(NBT,),
        in_specs=[
            pl.BlockSpec((BT, 1), lambda i: (i, 0)),
            pl.BlockSpec((BT, QL), lambda i: (i, 0)),
            pl.BlockSpec((IH, QL, ID - RD), lambda i: (0, 0, 0)),
            pl.BlockSpec((IH, QL, RD), lambda i: (0, 0, 0)),
        ],
        out_specs=[
            pl.BlockSpec((IH, BT, ID - RD), lambda i: (0, i, 0)),
            pl.BlockSpec((IH, BT, RD), lambda i: (0, i, 0)),
        ],
        out_shape=[
            jax.ShapeDtypeStruct((IH, T, ID - RD), jnp.float32),
            jax.ShapeDtypeStruct((IH, T, RD), jnp.float32),
        ],
    )(pos_col, qc, wia, wib)


def _k2b_body(kvc_ref, wkn_ref, wv_ref, kn_ref, v_ref):
    kvc = kvc_ref[...].astype(jnp.bfloat16)
    for h in range(H):
        kn = jnp.dot(kvc, wkn_ref[h], preferred_element_type=jnp.float32)
        kn_ref[h] = kn.astype(jnp.bfloat16)
        v = jnp.dot(kvc, wv_ref[h], preferred_element_type=jnp.float32)
        v_ref[h] = v.astype(jnp.bfloat16)


def _k2b(kvc, wkn, wv):
    return pl.pallas_call(
        _k2b_body,
        grid=(NBT,),
        in_specs=[
            pl.BlockSpec((BT, KVL), lambda i: (i, 0)),
            pl.BlockSpec((H, KVL, ND), lambda i: (0, 0, 0)),
            pl.BlockSpec((H, KVL, VD), lambda i: (0, 0, 0)),
        ],
        out_specs=[
            pl.BlockSpec((H, BT, ND), lambda i: (0, i, 0)),
            pl.BlockSpec((H, BT, VD), lambda i: (0, i, 0)),
        ],
        out_shape=[
            jax.ShapeDtypeStruct((H, T, ND), jnp.bfloat16),
            jax.ShapeDtypeStruct((H, T, VD), jnp.bfloat16),
        ],
    )(kvc, wkn, wv)


def _k3_body(iqa_ref, iqb_ref, ika_ref, ikb_ref, wts_ref, sc_ref, th_ref):
    i = pl.program_id(0)
    acc = jnp.zeros((BT, T), jnp.float32)
    ika = ika_ref[...]
    ikb = ikb_ref[...]
    for h in range(IH):
        lg = jax.lax.dot_general(iqa_ref[h], ika,
                                 (((1,), (1,)), ((), ())),
                                 preferred_element_type=jnp.float32)
        lg = lg + jax.lax.dot_general(iqb_ref[h], ikb,
                                      (((1,), (1,)), ((), ())),
                                      preferred_element_type=jnp.float32)
        w = wts_ref[...][:, h:h + 1]
        acc = acc + jnp.maximum(lg, 0.0) * w
    acc = acc * (ID ** -0.5)
    rows = i * BT + jax.lax.broadcasted_iota(jnp.int32, (BT, T), 0)
    cols = jax.lax.broadcasted_iota(jnp.int32, (BT, T), 1)
    sc = jnp.where(cols <= rows, acc, NEG)
    sc_ref[...] = sc
    b = jax.lax.bitcast_convert_type(sc, jnp.uint32)
    sign = jnp.uint32(0x80000000)
    keys = jnp.where(b >= sign, ~b, b | sign)
    lo = jnp.zeros((BT, 1), jnp.uint32)
    for bit in range(31, -1, -1):
        cand = lo | jnp.uint32(1 << bit)
        cnt = jnp.sum((keys >= cand).astype(jnp.int32), axis=1, keepdims=True)
        lo = jnp.where(cnt >= TOPK, cand, lo)
    tb = jnp.where(lo >= sign, lo ^ sign, ~lo)
    th_ref[...] = jax.lax.bitcast_convert_type(tb, jnp.float32)


def _k3(iqa, iqb, ika, ikb, wts):
    return pl.pallas_call(
        _k3_body,
        grid=(NBT,),
        in_specs=[
            pl.BlockSpec((IH, BT, ID - RD), lambda i: (0, i, 0)),
            pl.BlockSpec((IH, BT, RD), lambda i: (0, i, 0)),
            pl.BlockSpec((T, ID - RD), lambda i: (0, 0)),
            pl.BlockSpec((T, RD), lambda i: (0, 0)),
            pl.BlockSpec((BT, IH), lambda i: (i, 0)),
        ],
        out_specs=[
            pl.BlockSpec((BT, T), lambda i: (i, 0)),
            pl.BlockSpec((BT, 1), lambda i: (i, 0)),
        ],
        out_shape=[
            jax.ShapeDtypeStruct((T, T), jnp.float32),
            jax.ShapeDtypeStruct((T, 1), jnp.float32),
        ],
    )(iqa, iqb, ika, ikb, wts)


def _k5_body(qn_ref, qr_ref, kn_ref, kpe_ref, v_ref, sc_ref, th_ref, o_ref):
    i = pl.program_id(0)
    scale = (ND + RD) ** -0.5
    lg = jax.lax.dot_general(qn_ref[0], kn_ref[0], (((1,), (1,)), ((), ())),
                             preferred_element_type=jnp.float32)
    lg = lg + jax.lax.dot_general(qr_ref[0], kpe_ref[...],
                                  (((1,), (1,)), ((), ())),
                                  preferred_element_type=jnp.float32)
    lg = lg * scale
    rows = i * BT + jax.lax.broadcasted_iota(jnp.int32, (BT, T), 0)
    cols = jax.lax.broadcasted_iota(jnp.int32, (BT, T), 1)
    mask = (sc_ref[...] >= th_ref[...]) & (cols <= rows)
    lg = jnp.where(mask, lg, NEG)
    m = jnp.max(lg, axis=1, keepdims=True)
    e = jnp.exp(lg - m)
    p = (e / jnp.sum(e, axis=1, keepdims=True)).astype(jnp.bfloat16)
    o = jnp.dot(p, v_ref[0], preferred_element_type=jnp.float32)
    o_ref[0] = o.astype(jnp.bfloat16)


def _k5(qn, qr, kn, kpe, v, sc, th):
    return pl.pallas_call(
        _k5_body,
        grid=(NBT, H),
        in_specs=[
            pl.BlockSpec((1, BT, ND), lambda i, h: (h, i, 0)),
            pl.BlockSpec((1, BT, RD), lambda i, h: (h, i, 0)),
            pl.BlockSpec((1, T, ND), lambda i, h: (h, 0, 0)),
            pl.BlockSpec((T, RD), lambda i, h: (0, 0)),
            pl.BlockSpec((1, T, VD), lambda i, h: (h, 0, 0)),
            pl.BlockSpec((BT, T), lambda i, h: (i, 0)),
            pl.BlockSpec((BT, 1), lambda i, h: (i, 0)),
        ],
        out_specs=pl.BlockSpec((1, BT, VD), lambda i, h: (h, i, 0)),
        out_shape=jax.ShapeDtypeStruct((H, T, VD), jnp.bfloat16),
    )(qn, qr, kn, kpe, v, sc, th)


def _k6_body(ao_ref, wo_ref, res_ref, plw_ref, res2_ref, h2_ref):
    acc = jnp.zeros((BT, D), jnp.float32)
    for h in range(H):
        acc = acc + jnp.dot(ao_ref[h], wo_ref[h],
                            preferred_element_type=jnp.float32)
    res2 = acc + res_ref[...]
    res2_ref[...] = res2
    h2_ref[...] = _rms(res2, plw_ref[...])


def _k6(ao, wo_r, res, plw):
    return pl.pallas_call(
        _k6_body,
        grid=(NBT,),
        in_specs=[
            pl.BlockSpec((H, BT, VD), lambda i: (0, i, 0)),
            pl.BlockSpec((H, VD, D), lambda i: (0, 0, 0)),
            pl.BlockSpec((BT, D), lambda i: (i, 0)),
            pl.BlockSpec((1, D), lambda i: (0, 0)),
        ],
        out_specs=[
            pl.BlockSpec((BT, D), lambda i: (i, 0)),
            pl.BlockSpec((BT, D), lambda i: (i, 0)),
        ],
        out_shape=[
            jax.ShapeDtypeStruct((T, D), jnp.float32),
            jax.ShapeDtypeStruct((T, D), jnp.float32),
        ],
    )(ao, wo_r, res, plw.reshape(1, D))


def _k7_body(h2_ref, wg_ref, wu_ref, wd_ref, o_ref):
    @pl.when(pl.program_id(1) == 0)
    def _():
        o_ref[...] = jnp.zeros_like(o_ref)

    h2 = h2_ref[...].astype(jnp.bfloat16)
    g = jnp.dot(h2, wg_ref[...], preferred_element_type=jnp.float32)
    u = jnp.dot(h2, wu_ref[...], preferred_element_type=jnp.float32)
    a = (g * jax.lax.logistic(g) * u).astype(jnp.bfloat16)
    o_ref[...] += jnp.dot(a, wd_ref[...], preferred_element_type=jnp.float32)


def _k7(h2, wg, wu, wd):
    return pl.pallas_call(
        _k7_body,
        grid=(NBT, NBF),
        in_specs=[
            pl.BlockSpec((BT, D), lambda i, j: (i, 0)),
            pl.BlockSpec((D, BF), lambda i, j: (0, j)),
            pl.BlockSpec((D, BF), lambda i, j: (0, j)),
            pl.BlockSpec((BF, D), lambda i, j: (j, 0)),
        ],
        out_specs=pl.BlockSpec((BT, D), lambda i, j: (i, 0)),
        out_shape=jax.ShapeDtypeStruct((T, D), jnp.float32),
        compiler_params=pltpu.CompilerParams(
            dimension_semantics=("arbitrary", "arbitrary")),
    )(h2, wg, wu, wd)


def kernel(positions, hidden_states, residual, input_ln_w, post_ln_w, W_qkv_a,
           q_a_ln_w, kv_a_ln_w, W_q_b, W_idx_k, idx_k_norm_w, idx_k_norm_b,
           W_idx_wts, W_idx_q_b, W_kv_b, W_o, W_gate, W_up, W_down):
    pos_col = positions.astype(jnp.float32).reshape(T, 1)
    wq = W_q_b.reshape(QL, H, ND + RD)
    wqn = jnp.transpose(wq[:, :, :ND], (1, 0, 2)).astype(jnp.bfloat16)
    wqr = jnp.transpose(wq[:, :, ND:], (1, 0, 2)).astype(jnp.bfloat16)
    wi = W_idx_q_b.reshape(QL, IH, ID)
    wia = jnp.transpose(wi[:, :, :ID - RD], (1, 0, 2))
    wib = jnp.transpose(wi[:, :, ID - RD:], (1, 0, 2))
    wkv = W_kv_b.reshape(KVL, H, ND + VD)
    wkn = jnp.transpose(wkv[:, :, :ND], (1, 0, 2)).astype(jnp.bfloat16)
    wv = jnp.transpose(wkv[:, :, ND:], (1, 0, 2)).astype(jnp.bfloat16)
    wo_r = W_o.reshape(H, VD, D).astype(jnp.bfloat16)

    res, qc, kvc, kpe, ika, ikb, wts = _k1(
        pos_col, hidden_states, residual, input_ln_w, W_qkv_a, q_a_ln_w,
        kv_a_ln_w, W_idx_k, idx_k_norm_w, idx_k_norm_b, W_idx_wts)
    qn, qr = _k2a(pos_col, qc, wqn, wqr)
    iqa, iqb = _k2i(pos_col, qc, wia, wib)
    kn, v = _k2b(kvc, wkn, wv)
    sc, th = _k3(iqa, iqb, ika, ikb, wts)
    ao = _k5(qn, qr, kn, kpe, v, sc, th)
    res2, h2 = _k6(ao, wo_r, res, post_ln_w)
    mlp_out = _k7(h2, W_gate.astype(jnp.bfloat16),
                  W_up.astype(jnp.bfloat16), W_down.astype(jnp.bfloat16))
    return (mlp_out, res2)


# speedup vs baseline: 1.6983x; 1.0017x over previous
import functools

import jax
import jax.numpy as jnp
import numpy as np
from jax.experimental import pallas as pl
from jax.experimental.pallas import tpu as pltpu

T = 2048
D = 2048
H = 16
QL = 1536
KVL = 512
RD = 64
ND = 128
VD = 128
IH = 16
ID = 128
TOPK = 512
FF = 5632
EPS = 1e-6
NEG = -1e30

BT = 256
NBT = T // BT
BF = 1408
NBF = FF // BF


def _rope_cs(pos_col):
    j = jax.lax.broadcasted_iota(jnp.int32, (1, RD // 2), 1).astype(jnp.float32)
    inv = jnp.exp(j * (-np.log(10000.0) / (RD // 2)))
    f = pos_col * inv
    return jnp.cos(f), jnp.sin(f)


def _rope(x, cos, sin):
    x1 = x[:, : RD // 2]
    x2 = x[:, RD // 2:]
    return jnp.concatenate([x1 * cos - x2 * sin, x2 * cos + x1 * sin], axis=1)


def _rms(x, w):
    var = jnp.mean(x * x, axis=-1, keepdims=True)
    return x * jax.lax.rsqrt(var + EPS) * w


def _k1_body(pos_ref, hs_ref, rs_ref, ilw_ref, wqkv_ref, qalw_ref, kvalw_ref,
             widxk_ref, iknw_ref, iknb_ref, widxw_ref,
             res_ref, qc_ref, kvc_ref, kpe_ref, ika_ref, ikb_ref, wts_ref):
    h0 = hs_ref[...] + rs_ref[...]
    res_ref[...] = h0
    h = _rms(h0, ilw_ref[...])
    qkv = jnp.dot(h, wqkv_ref[...], preferred_element_type=jnp.float32)
    cos, sin = _rope_cs(pos_ref[...])
    qc_ref[...] = _rms(qkv[:, :QL], qalw_ref[...])
    kvc_ref[...] = _rms(qkv[:, QL:QL + KVL], kvalw_ref[...])
    kpe_ref[...] = _rope(qkv[:, QL + KVL:], cos, sin).astype(jnp.bfloat16)
    ik = jnp.dot(h, widxk_ref[...], preferred_element_type=jnp.float32)
    m = jnp.mean(ik, axis=-1, keepdims=True)
    v = jnp.mean((ik - m) ** 2, axis=-1, keepdims=True)
    ik = (ik - m) * jax.lax.rsqrt(v + EPS) * iknw_ref[...] + iknb_ref[...]
    ika_ref[...] = ik[:, :ID - RD]
    ikb_ref[...] = _rope(ik[:, ID - RD:], cos, sin)
    wts_ref[...] = jnp.dot(h, widxw_ref[...],
                           preferred_element_type=jnp.float32) * (IH ** -0.5)


def _k1(pos_col, hidden, resid, ilw, wqkv, qalw, kvalw, widxk, iknw, iknb, widxw):
    row = lambda i: (i, 0)
    fixed = lambda i: (0, 0)
    return pl.pallas_call(
        _k1_body,
        grid=(NBT,),
        in_specs=[
            pl.BlockSpec((BT, 1), row),
            pl.BlockSpec((BT, D), row),
            pl.BlockSpec((BT, D), row),
            pl.BlockSpec((1, D), fixed),
            pl.BlockSpec((D, QL + KVL + RD), fixed),
            pl.BlockSpec((1, QL), fixed),
            pl.BlockSpec((1, KVL), fixed),
            pl.BlockSpec((D, ID), fixed),
            pl.BlockSpec((1, ID), fixed),
            pl.BlockSpec((1, ID), fixed),
            pl.BlockSpec((D, IH), fixed),
        ],
        out_specs=[
            pl.BlockSpec((BT, D), row),
            pl.BlockSpec((BT, QL), row),
            pl.BlockSpec((BT, KVL), row),
            pl.BlockSpec((BT, RD), row),
            pl.BlockSpec((BT, ID - RD), row),
            pl.BlockSpec((BT, RD), row),
            pl.BlockSpec((BT, IH), row),
        ],
        out_shape=[
            jax.ShapeDtypeStruct((T, D), jnp.float32),
            jax.ShapeDtypeStruct((T, QL), jnp.float32),
            jax.ShapeDtypeStruct((T, KVL), jnp.float32),
            jax.ShapeDtypeStruct((T, RD), jnp.bfloat16),
            jax.ShapeDtypeStruct((T, ID - RD), jnp.float32),
            jax.ShapeDtypeStruct((T, RD), jnp.float32),
            jax.ShapeDtypeStruct((T, IH), jnp.float32),
        ],
    )(pos_col, hidden, resid, ilw.reshape(1, D), wqkv, qalw.reshape(1, QL),
      kvalw.reshape(1, KVL), widxk, iknw.reshape(1, ID), iknb.reshape(1, ID),
      widxw)


def _k2a_body(pos_ref, qc_ref, wqn_ref, wqr_ref, qn_ref, qr_ref):
    qc = qc_ref[...].astype(jnp.bfloat16)
    cos, sin = _rope_cs(pos_ref[...])
    qn = jnp.dot(qc, wqn_ref[...], preferred_element_type=jnp.float32)
    qn_ref[...] = qn.astype(jnp.bfloat16)
    for h in range(H):
        qr = jnp.dot(qc, wqr_ref[h], preferred_element_type=jnp.float32)
        qr_ref[h] = _rope(qr, cos, sin).astype(jnp.bfloat16)


def _k2a(pos_col, qc, wqn, wqr):
    return pl.pallas_call(
        _k2a_body,
        grid=(NBT,),
        in_specs=[
            pl.BlockSpec((BT, 1), lambda i: (i, 0)),
            pl.BlockSpec((BT, QL), lambda i: (i, 0)),
            pl.BlockSpec((QL, H * ND), lambda i: (0, 0)),
            pl.BlockSpec((H, QL, RD), lambda i: (0, 0, 0)),
        ],
        out_specs=[
            pl.BlockSpec((BT, H * ND), lambda i: (i, 0)),
            pl.BlockSpec((H, BT, RD), lambda i: (0, i, 0)),
        ],
        out_shape=[
            jax.ShapeDtypeStruct((T, H * ND), jnp.bfloat16),
            jax.ShapeDtypeStruct((H, T, RD), jnp.bfloat16),
        ],
    )(pos_col, qc, wqn, wqr)


def _k2i_body(pos_ref, qc_ref, wia_ref, wib_ref, iqa_ref, iqb_ref):
    qc = qc_ref[...]
    cos, sin = _rope_cs(pos_ref[...])
    for h in range(IH):
        iqa_ref[h] = jnp.dot(qc, wia_ref[h], preferred_element_type=jnp.float32)
        iqb = jnp.dot(qc, wib_ref[h], preferred_element_type=jnp.float32)
        iqb_ref[h] = _rope(iqb, cos, sin)


def _k2i(pos_col, qc, wia, wib):
    return pl.pallas_call(
        _k2i_body,
        grid=(NBT,),
        in_specs=[
            pl.BlockSpec((BT, 1), lambda i: (i, 0)),
            pl.BlockSpec((BT, QL), lambda i: (i, 0)),
            pl.BlockSpec((IH, QL, ID - RD), lambda i: (0, 0, 0)),
            pl.BlockSpec((IH, QL, RD), lambda i: (0, 0, 0)),
        ],
        out_specs=[
            pl.BlockSpec((IH, BT, ID - RD), lambda i: (0, i, 0)),
            pl.BlockSpec((IH, BT, RD), lambda i: (0, i, 0)),
        ],
        out_shape=[
            jax.ShapeDtypeStruct((IH, T, ID - RD), jnp.float32),
            jax.ShapeDtypeStruct((IH, T, RD), jnp.float32),
        ],
    )(pos_col, qc, wia, wib)


def _k2b_body(kvc_ref, wkn_ref, wv_ref, kn_ref, v_ref):
    kvc = kvc_ref[...].astype(jnp.bfloat16)
    kn = jnp.dot(kvc, wkn_ref[...], preferred_element_type=jnp.float32)
    kn_ref[...] = kn.astype(jnp.bfloat16)
    v = jnp.dot(kvc, wv_ref[...], preferred_element_type=jnp.float32)
    v_ref[...] = v.astype(jnp.bfloat16)


def _k2b(kvc, wkn, wv):
    return pl.pallas_call(
        _k2b_body,
        grid=(NBT,),
        in_specs=[
            pl.BlockSpec((BT, KVL), lambda i: (i, 0)),
            pl.BlockSpec((KVL, H * ND), lambda i: (0, 0)),
            pl.BlockSpec((KVL, H * VD), lambda i: (0, 0)),
        ],
        out_specs=[
            pl.BlockSpec((BT, H * ND), lambda i: (i, 0)),
            pl.BlockSpec((BT, H * VD), lambda i: (i, 0)),
        ],
        out_shape=[
            jax.ShapeDtypeStruct((T, H * ND), jnp.bfloat16),
            jax.ShapeDtypeStruct((T, H * VD), jnp.bfloat16),
        ],
    )(kvc, wkn, wv)


def _k3_body(iqa_ref, iqb_ref, ika_ref, ikb_ref, wts_ref, sc_ref, th_ref):
    i = pl.program_id(0)
    acc = jnp.zeros((BT, T), jnp.float32)
    ika = ika_ref[...]
    ikb = ikb_ref[...]
    for h in range(IH):
        lg = jax.lax.dot_general(iqa_ref[h], ika,
                                 (((1,), (1,)), ((), ())),
                                 preferred_element_type=jnp.float32)
        lg = lg + jax.lax.dot_general(iqb_ref[h], ikb,
                                      (((1,), (1,)), ((), ())),
                                      preferred_element_type=jnp.float32)
        w = wts_ref[...][:, h:h + 1]
        acc = acc + jnp.maximum(lg, 0.0) * w
    acc = acc * (ID ** -0.5)
    rows = i * BT + jax.lax.broadcasted_iota(jnp.int32, (BT, T), 0)
    cols = jax.lax.broadcasted_iota(jnp.int32, (BT, T), 1)
    sc = jnp.where(cols <= rows, acc, NEG)
    sc_ref[...] = sc
    b = jax.lax.bitcast_convert_type(sc, jnp.uint32)
    sign = jnp.uint32(0x80000000)
    keys = jnp.where(b >= sign, ~b, b | sign)
    lo = jnp.zeros((BT, 1), jnp.uint32)
    for bit in range(31, -1, -1):
        cand = lo | jnp.uint32(1 << bit)
        cnt = jnp.sum((keys >= cand).astype(jnp.int32), axis=1, keepdims=True)
        lo = jnp.where(cnt >= TOPK, cand, lo)
    tb = jnp.where(lo >= sign, lo ^ sign, ~lo)
    th_ref[...] = jax.lax.bitcast_convert_type(tb, jnp.float32)


def _k3(iqa, iqb, ika, ikb, wts):
    return pl.pallas_call(
        _k3_body,
        grid=(NBT,),
        in_specs=[
            pl.BlockSpec((IH, BT, ID - RD), lambda i: (0, i, 0)),
            pl.BlockSpec((IH, BT, RD), lambda i: (0, i, 0)),
            pl.BlockSpec((T, ID - RD), lambda i: (0, 0)),
            pl.BlockSpec((T, RD), lambda i: (0, 0)),
            pl.BlockSpec((BT, IH), lambda i: (i, 0)),
        ],
        out_specs=[
            pl.BlockSpec((BT, T), lambda i: (i, 0)),
            pl.BlockSpec((BT, 1), lambda i: (i, 0)),
        ],
        out_shape=[
            jax.ShapeDtypeStruct((T, T), jnp.float32),
            jax.ShapeDtypeStruct((T, 1), jnp.float32),
        ],
    )(iqa, iqb, ika, ikb, wts)


def _k5_body(qn_ref, qr_ref, kn_ref, kpe_ref, v_ref, sc_ref, th_ref,
             wo_ref, res_ref, plw_ref, res2_ref, h2_ref):
    i = pl.program_id(0)
    scale = (ND + RD) ** -0.5
    rows = i * BT + jax.lax.broadcasted_iota(jnp.int32, (BT, T), 0)
    cols = jax.lax.broadcasted_iota(jnp.int32, (BT, T), 1)
    mask = (sc_ref[...] >= th_ref[...]) & (cols <= rows)
    qn = qn_ref[...]
    kn = kn_ref[...]
    v = v_ref[...]
    kpe = kpe_ref[...]
    acc = jnp.zeros((BT, D), jnp.float32)
    for h in range(H):
        lg = jax.lax.dot_general(qn[:, h * ND:(h + 1) * ND],
                                 kn[:, h * ND:(h + 1) * ND],
                                 (((1,), (1,)), ((), ())),
                                 preferred_element_type=jnp.float32)
        lg = lg + jax.lax.dot_general(qr_ref[h], kpe,
                                      (((1,), (1,)), ((), ())),
                                      preferred_element_type=jnp.float32)
        lg = jnp.where(mask, lg * scale, NEG)
        m = jnp.max(lg, axis=1, keepdims=True)
        e = jnp.exp(lg - m)
        p = (e / jnp.sum(e, axis=1, keepdims=True)).astype(jnp.bfloat16)
        ao = jnp.dot(p, v[:, h * VD:(h + 1) * VD],
                     preferred_element_type=jnp.float32).astype(jnp.bfloat16)
        acc = acc + jnp.dot(ao, wo_ref[h], preferred_element_type=jnp.float32)
    res2 = acc + res_ref[...]
    res2_ref[...] = res2
    h2_ref[...] = _rms(res2, plw_ref[...])


def _k5(qn, qr, kn, kpe, v, sc, th, wo_r, res, plw):
    return pl.pallas_call(
        _k5_body,
        grid=(NBT,),
        in_specs=[
            pl.BlockSpec((BT, H * ND), lambda i: (i, 0)),
            pl.BlockSpec((H, BT, RD), lambda i: (0, i, 0)),
            pl.BlockSpec((T, H * ND), lambda i: (0, 0)),
            pl.BlockSpec((T, RD), lambda i: (0, 0)),
            pl.BlockSpec((T, H * VD), lambda i: (0, 0)),
            pl.BlockSpec((BT, T), lambda i: (i, 0)),
            pl.BlockSpec((BT, 1), lambda i: (i, 0)),
            pl.BlockSpec((H, VD, D), lambda i: (0, 0, 0)),
            pl.BlockSpec((BT, D), lambda i: (i, 0)),
            pl.BlockSpec((1, D), lambda i: (0, 0)),
        ],
        out_specs=[
            pl.BlockSpec((BT, D), lambda i: (i, 0)),
            pl.BlockSpec((BT, D), lambda i: (i, 0)),
        ],
        out_shape=[
            jax.ShapeDtypeStruct((T, D), jnp.float32),
            jax.ShapeDtypeStruct((T, D), jnp.float32),
        ],
    )(qn, qr, kn, kpe, v, sc, th, wo_r, res, plw.reshape(1, D))


def _k7_body(h2_ref, wg_ref, wu_ref, wd_ref, o_ref):
    @pl.when(pl.program_id(1) == 0)
    def _():
        o_ref[...] = jnp.zeros_like(o_ref)

    h2 = h2_ref[...].astype(jnp.bfloat16)
    g = jnp.dot(h2, wg_ref[...], preferred_element_type=jnp.float32)
    u = jnp.dot(h2, wu_ref[...], preferred_element_type=jnp.float32)
    a = (g * jax.lax.logistic(g) * u).astype(jnp.bfloat16)
    o_ref[...] += jnp.dot(a, wd_ref[...], preferred_element_type=jnp.float32)


def _k7(h2, wg, wu, wd):
    return pl.pallas_call(
        _k7_body,
        grid=(NBT, NBF),
        in_specs=[
            pl.BlockSpec((BT, D), lambda i, j: (i, 0)),
            pl.BlockSpec((D, BF), lambda i, j: (0, j)),
            pl.BlockSpec((D, BF), lambda i, j: (0, j)),
            pl.BlockSpec((BF, D), lambda i, j: (j, 0)),
        ],
        out_specs=pl.BlockSpec((BT, D), lambda i, j: (i, 0)),
        out_shape=jax.ShapeDtypeStruct((T, D), jnp.float32),
        compiler_params=pltpu.CompilerParams(
            dimension_semantics=("arbitrary", "arbitrary")),
    )(h2, wg, wu, wd)


def kernel(positions, hidden_states, residual, input_ln_w, post_ln_w, W_qkv_a,
           q_a_ln_w, kv_a_ln_w, W_q_b, W_idx_k, idx_k_norm_w, idx_k_norm_b,
           W_idx_wts, W_idx_q_b, W_kv_b, W_o, W_gate, W_up, W_down):
    pos_col = positions.astype(jnp.float32).reshape(T, 1)
    wq = W_q_b.reshape(QL, H, ND + RD)
    wqn = wq[:, :, :ND].reshape(QL, H * ND).astype(jnp.bfloat16)
    wqr = jnp.transpose(wq[:, :, ND:], (1, 0, 2)).astype(jnp.bfloat16)
    wi = W_idx_q_b.reshape(QL, IH, ID)
    wia = jnp.transpose(wi[:, :, :ID - RD], (1, 0, 2))
    wib = jnp.transpose(wi[:, :, ID - RD:], (1, 0, 2))
    wkv = W_kv_b.reshape(KVL, H, ND + VD)
    wkn = wkv[:, :, :ND].reshape(KVL, H * ND).astype(jnp.bfloat16)
    wv = wkv[:, :, ND:].reshape(KVL, H * VD).astype(jnp.bfloat16)
    wo_r = W_o.reshape(H, VD, D).astype(jnp.bfloat16)

    res, qc, kvc, kpe, ika, ikb, wts = _k1(
        pos_col, hidden_states, residual, input_ln_w, W_qkv_a, q_a_ln_w,
        kv_a_ln_w, W_idx_k, idx_k_norm_w, idx_k_norm_b, W_idx_wts)
    qn, qr = _k2a(pos_col, qc, wqn, wqr)
    iqa, iqb = _k2i(pos_col, qc, wia, wib)
    kn, v = _k2b(kvc, wkn, wv)
    sc, th = _k3(iqa, iqb, ika, ikb, wts)
    res2, h2 = _k5(qn, qr, kn, kpe, v, sc, th, wo_r, res, post_ln_w)
    mlp_out = _k7(h2, W_gate.astype(jnp.bfloat16),
                  W_up.astype(jnp.bfloat16), W_down.astype(jnp.bfloat16))
    return (mlp_out, res2)


# speedup vs baseline: 1.8803x; 1.1072x over previous
import functools

import jax
import jax.numpy as jnp
import numpy as np
from jax.experimental import pallas as pl
from jax.experimental.pallas import tpu as pltpu

T = 2048
D = 2048
H = 16
QL = 1536
KVL = 512
RD = 64
ND = 128
VD = 128
IH = 16
ID = 128
TOPK = 512
FF = 5632
EPS = 1e-6
NEG = -1e30

BT = 256
NBT = T // BT
BF = 256
NBF = FF // BF


def _rope_cs(pos_col):
    j = jax.lax.broadcasted_iota(jnp.int32, (1, RD // 2), 1).astype(jnp.float32)
    inv = jnp.exp(j * (-np.log(10000.0) / (RD // 2)))
    f = pos_col * inv
    return jnp.cos(f), jnp.sin(f)


def _rope(x, cos, sin):
    x1 = x[:, : RD // 2]
    x2 = x[:, RD // 2:]
    return jnp.concatenate([x1 * cos - x2 * sin, x2 * cos + x1 * sin], axis=1)


def _rms(x, w):
    var = jnp.mean(x * x, axis=-1, keepdims=True)
    return x * jax.lax.rsqrt(var + EPS) * w


def _k1_body(pos_ref, hs_ref, rs_ref, ilw_ref, wqkv_ref, qalw_ref, kvalw_ref,
             widxk_ref, iknw_ref, iknb_ref, widxw_ref,
             res_ref, qc_ref, kvc_ref, kpe_ref, ika_ref, ikb_ref, wts_ref):
    h0 = hs_ref[...] + rs_ref[...]
    res_ref[...] = h0
    h = _rms(h0, ilw_ref[...])
    qkv = jnp.dot(h, wqkv_ref[...], preferred_element_type=jnp.float32)
    cos, sin = _rope_cs(pos_ref[...])
    qc_ref[...] = _rms(qkv[:, :QL], qalw_ref[...])
    kvc_ref[...] = _rms(qkv[:, QL:QL + KVL], kvalw_ref[...])
    kpe_ref[...] = _rope(qkv[:, QL + KVL:], cos, sin).astype(jnp.bfloat16)
    ik = jnp.dot(h, widxk_ref[...], preferred_element_type=jnp.float32)
    m = jnp.mean(ik, axis=-1, keepdims=True)
    v = jnp.mean((ik - m) ** 2, axis=-1, keepdims=True)
    ik = (ik - m) * jax.lax.rsqrt(v + EPS) * iknw_ref[...] + iknb_ref[...]
    ika_ref[...] = ik[:, :ID - RD]
    ikb_ref[...] = _rope(ik[:, ID - RD:], cos, sin)
    wts_ref[...] = jnp.dot(h, widxw_ref[...],
                           preferred_element_type=jnp.float32) * (IH ** -0.5)


def _k1(pos_col, hidden, resid, ilw, wqkv, qalw, kvalw, widxk, iknw, iknb, widxw):
    row = lambda i: (i, 0)
    fixed = lambda i: (0, 0)
    return pl.pallas_call(
        _k1_body,
        grid=(NBT,),
        in_specs=[
            pl.BlockSpec((BT, 1), row),
            pl.BlockSpec((BT, D), row),
            pl.BlockSpec((BT, D), row),
            pl.BlockSpec((1, D), fixed),
            pl.BlockSpec((D, QL + KVL + RD), fixed),
            pl.BlockSpec((1, QL), fixed),
            pl.BlockSpec((1, KVL), fixed),
            pl.BlockSpec((D, ID), fixed),
            pl.BlockSpec((1, ID), fixed),
            pl.BlockSpec((1, ID), fixed),
            pl.BlockSpec((D, IH), fixed),
        ],
        out_specs=[
            pl.BlockSpec((BT, D), row),
            pl.BlockSpec((BT, QL), row),
            pl.BlockSpec((BT, KVL), row),
            pl.BlockSpec((BT, RD), row),
            pl.BlockSpec((BT, ID - RD), row),
            pl.BlockSpec((BT, RD), row),
            pl.BlockSpec((BT, IH), row),
        ],
        out_shape=[
            jax.ShapeDtypeStruct((T, D), jnp.float32),
            jax.ShapeDtypeStruct((T, QL), jnp.float32),
            jax.ShapeDtypeStruct((T, KVL), jnp.float32),
            jax.ShapeDtypeStruct((T, RD), jnp.bfloat16),
            jax.ShapeDtypeStruct((T, ID - RD), jnp.float32),
            jax.ShapeDtypeStruct((T, RD), jnp.float32),
            jax.ShapeDtypeStruct((T, IH), jnp.float32),
        ],
    )(pos_col, hidden, resid, ilw.reshape(1, D), wqkv, qalw.reshape(1, QL),
      kvalw.reshape(1, KVL), widxk, iknw.reshape(1, ID), iknb.reshape(1, ID),
      widxw)


def _k2a_body(pos_ref, qc_ref, wqn_ref, wqr_ref, qn_ref, qr_ref):
    qc = qc_ref[...].astype(jnp.bfloat16)
    cos, sin = _rope_cs(pos_ref[...])
    qn = jnp.dot(qc, wqn_ref[...], preferred_element_type=jnp.float32)
    qn_ref[...] = qn.astype(jnp.bfloat16)
    for h in range(H):
        qr = jnp.dot(qc, wqr_ref[h], preferred_element_type=jnp.float32)
        qr_ref[h] = _rope(qr, cos, sin).astype(jnp.bfloat16)


def _k2a(pos_col, qc, wqn, wqr):
    return pl.pallas_call(
        _k2a_body,
        grid=(NBT,),
        in_specs=[
            pl.BlockSpec((BT, 1), lambda i: (i, 0)),
            pl.BlockSpec((BT, QL), lambda i: (i, 0)),
            pl.BlockSpec((QL, H * ND), lambda i: (0, 0)),
            pl.BlockSpec((H, QL, RD), lambda i: (0, 0, 0)),
        ],
        out_specs=[
            pl.BlockSpec((BT, H * ND), lambda i: (i, 0)),
            pl.BlockSpec((H, BT, RD), lambda i: (0, i, 0)),
        ],
        out_shape=[
            jax.ShapeDtypeStruct((T, H * ND), jnp.bfloat16),
            jax.ShapeDtypeStruct((H, T, RD), jnp.bfloat16),
        ],
    )(pos_col, qc, wqn, wqr)


def _k2i_body(pos_ref, qc_ref, wia_ref, wib_ref, iqa_ref, iqb_ref):
    qc = qc_ref[...]
    cos, sin = _rope_cs(pos_ref[...])
    for h in range(IH):
        iqa_ref[h] = jnp.dot(qc, wia_ref[h], preferred_element_type=jnp.float32)
        iqb = jnp.dot(qc, wib_ref[h], preferred_element_type=jnp.float32)
        iqb_ref[h] = _rope(iqb, cos, sin)


def _k2i(pos_col, qc, wia, wib):
    return pl.pallas_call(
        _k2i_body,
        grid=(NBT,),
        in_specs=[
            pl.BlockSpec((BT, 1), lambda i: (i, 0)),
            pl.BlockSpec((BT, QL), lambda i: (i, 0)),
            pl.BlockSpec((IH, QL, ID - RD), lambda i: (0, 0, 0)),
            pl.BlockSpec((IH, QL, RD), lambda i: (0, 0, 0)),
        ],
        out_specs=[
            pl.BlockSpec((IH, BT, ID - RD), lambda i: (0, i, 0)),
            pl.BlockSpec((IH, BT, RD), lambda i: (0, i, 0)),
        ],
        out_shape=[
            jax.ShapeDtypeStruct((IH, T, ID - RD), jnp.float32),
            jax.ShapeDtypeStruct((IH, T, RD), jnp.float32),
        ],
    )(pos_col, qc, wia, wib)


def _k2b_body(kvc_ref, wkn_ref, wv_ref, kn_ref, v_ref):
    kvc = kvc_ref[...].astype(jnp.bfloat16)
    kn = jnp.dot(kvc, wkn_ref[...], preferred_element_type=jnp.float32)
    kn_ref[...] = kn.astype(jnp.bfloat16)
    v = jnp.dot(kvc, wv_ref[...], preferred_element_type=jnp.float32)
    v_ref[...] = v.astype(jnp.bfloat16)


def _k2b(kvc, wkn, wv):
    return pl.pallas_call(
        _k2b_body,
        grid=(NBT,),
        in_specs=[
            pl.BlockSpec((BT, KVL), lambda i: (i, 0)),
            pl.BlockSpec((KVL, H * ND), lambda i: (0, 0)),
            pl.BlockSpec((KVL, H * VD), lambda i: (0, 0)),
        ],
        out_specs=[
            pl.BlockSpec((BT, H * ND), lambda i: (i, 0)),
            pl.BlockSpec((BT, H * VD), lambda i: (i, 0)),
        ],
        out_shape=[
            jax.ShapeDtypeStruct((T, H * ND), jnp.bfloat16),
            jax.ShapeDtypeStruct((T, H * VD), jnp.bfloat16),
        ],
    )(kvc, wkn, wv)


def _k3_body(iqa_ref, iqb_ref, ika_ref, ikb_ref, wts_ref, sc_ref, th_ref):
    i = pl.program_id(0)
    acc = jnp.zeros((BT, T), jnp.float32)
    ika = ika_ref[...]
    ikb = ikb_ref[...]
    for h in range(IH):
        lg = jax.lax.dot_general(iqa_ref[h], ika,
                                 (((1,), (1,)), ((), ())),
                                 preferred_element_type=jnp.float32)
        lg = lg + jax.lax.dot_general(iqb_ref[h], ikb,
                                      (((1,), (1,)), ((), ())),
                                      preferred_element_type=jnp.float32)
        w = wts_ref[...][:, h:h + 1]
        acc = acc + jnp.maximum(lg, 0.0) * w
    acc = acc * (ID ** -0.5)
    rows = i * BT + jax.lax.broadcasted_iota(jnp.int32, (BT, T), 0)
    cols = jax.lax.broadcasted_iota(jnp.int32, (BT, T), 1)
    sc = jnp.where(cols <= rows, acc, NEG)
    sc_ref[...] = sc
    b = jax.lax.bitcast_convert_type(sc, jnp.uint32)
    sign = jnp.uint32(0x80000000)
    keys = jnp.where(b >= sign, ~b, b | sign)
    lo = jnp.zeros((BT, 1), jnp.uint32)
    for bit in range(31, -1, -1):
        cand = lo | jnp.uint32(1 << bit)
        cnt = jnp.sum((keys >= cand).astype(jnp.int32), axis=1, keepdims=True)
        lo = jnp.where(cnt >= TOPK, cand, lo)
    tb = jnp.where(lo >= sign, lo ^ sign, ~lo)
    th_ref[...] = jax.lax.bitcast_convert_type(tb, jnp.float32)


def _k3(iqa, iqb, ika, ikb, wts):
    return pl.pallas_call(
        _k3_body,
        grid=(NBT,),
        in_specs=[
            pl.BlockSpec((IH, BT, ID - RD), lambda i: (0, i, 0)),
            pl.BlockSpec((IH, BT, RD), lambda i: (0, i, 0)),
            pl.BlockSpec((T, ID - RD), lambda i: (0, 0)),
            pl.BlockSpec((T, RD), lambda i: (0, 0)),
            pl.BlockSpec((BT, IH), lambda i: (i, 0)),
        ],
        out_specs=[
            pl.BlockSpec((BT, T), lambda i: (i, 0)),
            pl.BlockSpec((BT, 1), lambda i: (i, 0)),
        ],
        out_shape=[
            jax.ShapeDtypeStruct((T, T), jnp.float32),
            jax.ShapeDtypeStruct((T, 1), jnp.float32),
        ],
    )(iqa, iqb, ika, ikb, wts)


def _k5_body(qn_ref, qr_ref, kn_ref, kpe_ref, v_ref, sc_ref, th_ref,
             wo_ref, res_ref, plw_ref, res2_ref, h2_ref):
    i = pl.program_id(0)
    scale = (ND + RD) ** -0.5
    rows = i * BT + jax.lax.broadcasted_iota(jnp.int32, (BT, T), 0)
    cols = jax.lax.broadcasted_iota(jnp.int32, (BT, T), 1)
    mask = (sc_ref[...] >= th_ref[...]) & (cols <= rows)
    qn = qn_ref[...]
    kn = kn_ref[...]
    v = v_ref[...]
    kpe = kpe_ref[...]
    acc = jnp.zeros((BT, D), jnp.float32)
    for h in range(H):
        lg = jax.lax.dot_general(qn[:, h * ND:(h + 1) * ND],
                                 kn[:, h * ND:(h + 1) * ND],
                                 (((1,), (1,)), ((), ())),
                                 preferred_element_type=jnp.float32)
        lg = lg + jax.lax.dot_general(qr_ref[h], kpe,
                                      (((1,), (1,)), ((), ())),
                                      preferred_element_type=jnp.float32)
        lg = jnp.where(mask, lg * scale, NEG)
        m = jnp.max(lg, axis=1, keepdims=True)
        e = jnp.exp(lg - m)
        p = (e / jnp.sum(e, axis=1, keepdims=True)).astype(jnp.bfloat16)
        ao = jnp.dot(p, v[:, h * VD:(h + 1) * VD],
                     preferred_element_type=jnp.float32).astype(jnp.bfloat16)
        acc = acc + jnp.dot(ao, wo_ref[h], preferred_element_type=jnp.float32)
    res2 = acc + res_ref[...]
    res2_ref[...] = res2
    h2_ref[...] = _rms(res2, plw_ref[...]).astype(jnp.bfloat16)


def _k5(qn, qr, kn, kpe, v, sc, th, wo_r, res, plw):
    return pl.pallas_call(
        _k5_body,
        grid=(NBT,),
        in_specs=[
            pl.BlockSpec((BT, H * ND), lambda i: (i, 0)),
            pl.BlockSpec((H, BT, RD), lambda i: (0, i, 0)),
            pl.BlockSpec((T, H * ND), lambda i: (0, 0)),
            pl.BlockSpec((T, RD), lambda i: (0, 0)),
            pl.BlockSpec((T, H * VD), lambda i: (0, 0)),
            pl.BlockSpec((BT, T), lambda i: (i, 0)),
            pl.BlockSpec((BT, 1), lambda i: (i, 0)),
            pl.BlockSpec((H, VD, D), lambda i: (0, 0, 0)),
            pl.BlockSpec((BT, D), lambda i: (i, 0)),
            pl.BlockSpec((1, D), lambda i: (0, 0)),
        ],
        out_specs=[
            pl.BlockSpec((BT, D), lambda i: (i, 0)),
            pl.BlockSpec((BT, D), lambda i: (i, 0)),
        ],
        out_shape=[
            jax.ShapeDtypeStruct((T, D), jnp.float32),
            jax.ShapeDtypeStruct((T, D), jnp.bfloat16),
        ],
    )(qn, qr, kn, kpe, v, sc, th, wo_r, res, plw.reshape(1, D))


def _k7_body(h2_ref, wg_ref, wu_ref, wd_ref, o_ref):
    @pl.when(pl.program_id(0) == 0)
    def _():
        o_ref[...] = jnp.zeros_like(o_ref)

    h2 = h2_ref[...]
    wg = wg_ref[...].astype(jnp.bfloat16)
    wu = wu_ref[...].astype(jnp.bfloat16)
    wd = wd_ref[...].astype(jnp.bfloat16)
    g = jnp.dot(h2, wg, preferred_element_type=jnp.float32)
    u = jnp.dot(h2, wu, preferred_element_type=jnp.float32)
    a = (g * jax.lax.logistic(g) * u).astype(jnp.bfloat16)
    o_ref[...] += jnp.dot(a, wd, preferred_element_type=jnp.float32)


def _k7(h2, wg, wu, wd):
    return pl.pallas_call(
        _k7_body,
        grid=(NBF,),
        in_specs=[
            pl.BlockSpec((T, D), lambda j: (0, 0)),
            pl.BlockSpec((D, BF), lambda j: (0, j)),
            pl.BlockSpec((D, BF), lambda j: (0, j)),
            pl.BlockSpec((BF, D), lambda j: (j, 0)),
        ],
        out_specs=pl.BlockSpec((T, D), lambda j: (0, 0)),
        out_shape=jax.ShapeDtypeStruct((T, D), jnp.float32),
        compiler_params=pltpu.CompilerParams(
            dimension_semantics=("arbitrary",)),
    )(h2, wg, wu, wd)


def kernel(positions, hidden_states, residual, input_ln_w, post_ln_w, W_qkv_a,
           q_a_ln_w, kv_a_ln_w, W_q_b, W_idx_k, idx_k_norm_w, idx_k_norm_b,
           W_idx_wts, W_idx_q_b, W_kv_b, W_o, W_gate, W_up, W_down):
    pos_col = positions.astype(jnp.float32).reshape(T, 1)
    wq = W_q_b.reshape(QL, H, ND + RD)
    wqn = wq[:, :, :ND].reshape(QL, H * ND).astype(jnp.bfloat16)
    wqr = jnp.transpose(wq[:, :, ND:], (1, 0, 2)).astype(jnp.bfloat16)
    wi = W_idx_q_b.reshape(QL, IH, ID)
    wia = jnp.transpose(wi[:, :, :ID - RD], (1, 0, 2))
    wib = jnp.transpose(wi[:, :, ID - RD:], (1, 0, 2))
    wkv = W_kv_b.reshape(KVL, H, ND + VD)
    wkn = wkv[:, :, :ND].reshape(KVL, H * ND).astype(jnp.bfloat16)
    wv = wkv[:, :, ND:].reshape(KVL, H * VD).astype(jnp.bfloat16)
    wo_r = W_o.reshape(H, VD, D).astype(jnp.bfloat16)

    res, qc, kvc, kpe, ika, ikb, wts = _k1(
        pos_col, hidden_states, residual, input_ln_w, W_qkv_a, q_a_ln_w,
        kv_a_ln_w, W_idx_k, idx_k_norm_w, idx_k_norm_b, W_idx_wts)
    qn, qr = _k2a(pos_col, qc, wqn, wqr)
    iqa, iqb = _k2i(pos_col, qc, wia, wib)
    kn, v = _k2b(kvc, wkn, wv)
    sc, th = _k3(iqa, iqb, ika, ikb, wts)
    res2, h2 = _k5(qn, qr, kn, kpe, v, sc, th, wo_r, res, post_ln_w)
    mlp_out = _k7(h2, W_gate, W_up, W_down)
    return (mlp_out, res2)


# speedup vs baseline: 2.1458x; 1.1412x over previous
import functools

import jax
import jax.numpy as jnp
import numpy as np
from jax.experimental import pallas as pl
from jax.experimental.pallas import tpu as pltpu

T = 2048
D = 2048
H = 16
QL = 1536
KVL = 512
RD = 64
ND = 128
VD = 128
IH = 16
ID = 128
TOPK = 512
FF = 5632
EPS = 1e-6
NEG = -1e30

BT = 256
NBT = T // BT
BF = 256
NBF = FF // BF


def _rope_cs(pos_col):
    j = jax.lax.broadcasted_iota(jnp.int32, (1, RD // 2), 1).astype(jnp.float32)
    inv = jnp.exp(j * (-np.log(10000.0) / (RD // 2)))
    f = pos_col * inv
    return jnp.cos(f), jnp.sin(f)


def _rope(x, cos, sin):
    x1 = x[:, : RD // 2]
    x2 = x[:, RD // 2:]
    return jnp.concatenate([x1 * cos - x2 * sin, x2 * cos + x1 * sin], axis=1)


def _rms(x, w):
    var = jnp.mean(x * x, axis=-1, keepdims=True)
    return x * jax.lax.rsqrt(var + EPS) * w


def _k1_body(pos_ref, hs_ref, rs_ref, ilw_ref, wqkv_ref, qalw_ref, kvalw_ref,
             widxk_ref, iknw_ref, iknb_ref, widxw_ref,
             res_ref, qc_ref, kvc_ref, kpe_ref, ika_ref, wts_ref):
    h0 = hs_ref[...] + rs_ref[...]
    res_ref[...] = h0
    h = _rms(h0, ilw_ref[...])
    qkv = jnp.dot(h, wqkv_ref[...], preferred_element_type=jnp.float32)
    cos, sin = _rope_cs(pos_ref[...])
    qc_ref[...] = _rms(qkv[:, :QL], qalw_ref[...])
    kvc_ref[...] = _rms(qkv[:, QL:QL + KVL], kvalw_ref[...])
    kpe_ref[...] = _rope(qkv[:, QL + KVL:], cos, sin).astype(jnp.bfloat16)
    ik = jnp.dot(h, widxk_ref[...], preferred_element_type=jnp.float32)
    m = jnp.mean(ik, axis=-1, keepdims=True)
    v = jnp.mean((ik - m) ** 2, axis=-1, keepdims=True)
    ik = (ik - m) * jax.lax.rsqrt(v + EPS) * iknw_ref[...] + iknb_ref[...]
    ika_ref[...] = jnp.concatenate(
        [ik[:, :ID - RD], _rope(ik[:, ID - RD:], cos, sin)], axis=1)
    wts_ref[...] = jnp.dot(h, widxw_ref[...],
                           preferred_element_type=jnp.float32) * (IH ** -0.5)


def _k1(pos_col, hidden, resid, ilw, wqkv, qalw, kvalw, widxk, iknw, iknb, widxw):
    row = lambda i: (i, 0)
    fixed = lambda i: (0, 0)
    return pl.pallas_call(
        _k1_body,
        grid=(NBT,),
        in_specs=[
            pl.BlockSpec((BT, 1), row),
            pl.BlockSpec((BT, D), row),
            pl.BlockSpec((BT, D), row),
            pl.BlockSpec((1, D), fixed),
            pl.BlockSpec((D, QL + KVL + RD), fixed),
            pl.BlockSpec((1, QL), fixed),
            pl.BlockSpec((1, KVL), fixed),
            pl.BlockSpec((D, ID), fixed),
            pl.BlockSpec((1, ID), fixed),
            pl.BlockSpec((1, ID), fixed),
            pl.BlockSpec((D, IH), fixed),
        ],
        out_specs=[
            pl.BlockSpec((BT, D), row),
            pl.BlockSpec((BT, QL), row),
            pl.BlockSpec((BT, KVL), row),
            pl.BlockSpec((BT, RD), row),
            pl.BlockSpec((BT, ID), row),
            pl.BlockSpec((BT, IH), row),
        ],
        out_shape=[
            jax.ShapeDtypeStruct((T, D), jnp.float32),
            jax.ShapeDtypeStruct((T, QL), jnp.float32),
            jax.ShapeDtypeStruct((T, KVL), jnp.float32),
            jax.ShapeDtypeStruct((T, RD), jnp.bfloat16),
            jax.ShapeDtypeStruct((T, ID), jnp.float32),
            jax.ShapeDtypeStruct((T, IH), jnp.float32),
        ],
    )(pos_col, hidden, resid, ilw.reshape(1, D), wqkv, qalw.reshape(1, QL),
      kvalw.reshape(1, KVL), widxk, iknw.reshape(1, ID), iknb.reshape(1, ID),
      widxw)


def _k2a_body(pos_ref, qc_ref, wqn_ref, wqr_ref, qn_ref, qr_ref):
    qc = qc_ref[...].astype(jnp.bfloat16)
    cos, sin = _rope_cs(pos_ref[...])
    qn = jnp.dot(qc, wqn_ref[...], preferred_element_type=jnp.float32)
    qn_ref[...] = qn.astype(jnp.bfloat16)
    for h in range(H):
        qr = jnp.dot(qc, wqr_ref[h], preferred_element_type=jnp.float32)
        qr_ref[h] = _rope(qr, cos, sin).astype(jnp.bfloat16)


def _k2a(pos_col, qc, wqn, wqr):
    return pl.pallas_call(
        _k2a_body,
        grid=(NBT,),
        in_specs=[
            pl.BlockSpec((BT, 1), lambda i: (i, 0)),
            pl.BlockSpec((BT, QL), lambda i: (i, 0)),
            pl.BlockSpec((QL, H * ND), lambda i: (0, 0)),
            pl.BlockSpec((H, QL, RD), lambda i: (0, 0, 0)),
        ],
        out_specs=[
            pl.BlockSpec((BT, H * ND), lambda i: (i, 0)),
            pl.BlockSpec((H, BT, RD), lambda i: (0, i, 0)),
        ],
        out_shape=[
            jax.ShapeDtypeStruct((T, H * ND), jnp.bfloat16),
            jax.ShapeDtypeStruct((H, T, RD), jnp.bfloat16),
        ],
    )(pos_col, qc, wqn, wqr)


def _k2i_body(pos_ref, qc_ref, wi_ref, iq_ref):
    qc = qc_ref[...]
    cos, sin = _rope_cs(pos_ref[...])
    for h in range(IH):
        iq = jnp.dot(qc, wi_ref[h], preferred_element_type=jnp.float32)
        iq_ref[h] = jnp.concatenate(
            [iq[:, :ID - RD], _rope(iq[:, ID - RD:], cos, sin)], axis=1)


def _k2i(pos_col, qc, wi):
    return pl.pallas_call(
        _k2i_body,
        grid=(NBT,),
        in_specs=[
            pl.BlockSpec((BT, 1), lambda i: (i, 0)),
            pl.BlockSpec((BT, QL), lambda i: (i, 0)),
            pl.BlockSpec((IH, QL, ID), lambda i: (0, 0, 0)),
        ],
        out_specs=pl.BlockSpec((IH, BT, ID), lambda i: (0, i, 0)),
        out_shape=jax.ShapeDtypeStruct((IH, T, ID), jnp.float32),
    )(pos_col, qc, wi)


def _k2b_body(kvc_ref, wkn_ref, wv_ref, kn_ref, v_ref):
    kvc = kvc_ref[...].astype(jnp.bfloat16)
    kn = jnp.dot(kvc, wkn_ref[...], preferred_element_type=jnp.float32)
    kn_ref[...] = kn.astype(jnp.bfloat16)
    v = jnp.dot(kvc, wv_ref[...], preferred_element_type=jnp.float32)
    v_ref[...] = v.astype(jnp.bfloat16)


def _k2b(kvc, wkn, wv):
    return pl.pallas_call(
        _k2b_body,
        grid=(NBT,),
        in_specs=[
            pl.BlockSpec((BT, KVL), lambda i: (i, 0)),
            pl.BlockSpec((KVL, H * ND), lambda i: (0, 0)),
            pl.BlockSpec((KVL, H * VD), lambda i: (0, 0)),
        ],
        out_specs=[
            pl.BlockSpec((BT, H * ND), lambda i: (i, 0)),
            pl.BlockSpec((BT, H * VD), lambda i: (i, 0)),
        ],
        out_shape=[
            jax.ShapeDtypeStruct((T, H * ND), jnp.bfloat16),
            jax.ShapeDtypeStruct((T, H * VD), jnp.bfloat16),
        ],
    )(kvc, wkn, wv)


GR = 512
NG = T // GR

NEGKEY = int(~np.array(NEG, dtype=np.float32).view(np.uint32)) & 0xFFFFFFFF


def _k3_body(iq_ref, ik_ref, wts_ref, sc_ref, th_ref, *, g, W):
    i = pl.program_id(0)
    acc = jnp.zeros((BT, W), jnp.float32)
    ik = ik_ref[...]
    for h in range(IH):
        lg = jax.lax.dot_general(iq_ref[h], ik,
                                 (((1,), (1,)), ((), ())),
                                 preferred_element_type=jnp.float32)
        w = wts_ref[...][:, h:h + 1]
        acc = acc + jnp.maximum(lg, 0.0) * w
    acc = acc * (ID ** -0.5)
    rows = g * GR + i * BT + jax.lax.broadcasted_iota(jnp.int32, (BT, W), 0)
    cols = jax.lax.broadcasted_iota(jnp.int32, (BT, W), 1)
    sc = jnp.where(cols <= rows, acc, NEG)
    sc_ref[...] = sc
    b = jax.lax.bitcast_convert_type(sc, jnp.uint32)
    sign = jnp.uint32(0x80000000)
    keys = jnp.where(b >= sign, ~b, b | sign)
    lo = jnp.zeros((BT, 1), jnp.uint32)
    nfill = jnp.int32(T - W)
    for bit in range(31, -1, -1):
        cand = lo | jnp.uint32(1 << bit)
        cnt = jnp.sum((keys >= cand).astype(jnp.int32), axis=1, keepdims=True)
        cnt = cnt + jnp.where(jnp.uint32(NEGKEY) >= cand, nfill, 0)
        lo = jnp.where(cnt >= TOPK, cand, lo)
    tb = jnp.where(lo >= sign, lo ^ sign, ~lo)
    th_ref[...] = jax.lax.bitcast_convert_type(tb, jnp.float32)


def _k3(iq, ik, wts, g):
    W = (g + 1) * GR
    r0 = g * (GR // BT)
    import functools as _ft
    return pl.pallas_call(
        _ft.partial(_k3_body, g=g, W=W),
        grid=(GR // BT,),
        in_specs=[
            pl.BlockSpec((IH, BT, ID), lambda i: (0, r0 + i, 0)),
            pl.BlockSpec((W, ID), lambda i: (0, 0)),
            pl.BlockSpec((BT, IH), lambda i: (r0 + i, 0)),
        ],
        out_specs=[
            pl.BlockSpec((BT, W), lambda i: (i, 0)),
            pl.BlockSpec((BT, 1), lambda i: (i, 0)),
        ],
        out_shape=[
            jax.ShapeDtypeStruct((GR, W), jnp.float32),
            jax.ShapeDtypeStruct((GR, 1), jnp.float32),
        ],
    )(iq, ik, wts)


def _k5_body(qn_ref, qr_ref, kn_ref, kpe_ref, v_ref, sc_ref, th_ref,
             wo_ref, res_ref, plw_ref, res2_ref, h2_ref, *, g, W):
    i = pl.program_id(0)
    scale = (ND + RD) ** -0.5
    rows = g * GR + i * BT + jax.lax.broadcasted_iota(jnp.int32, (BT, W), 0)
    cols = jax.lax.broadcasted_iota(jnp.int32, (BT, W), 1)
    mask = (sc_ref[...] >= th_ref[...]) & (cols <= rows)
    qn = qn_ref[...]
    kn = kn_ref[...]
    v = v_ref[...]
    kpe = kpe_ref[...]
    acc = jnp.zeros((BT, D), jnp.float32)
    for h in range(H):
        lg = jax.lax.dot_general(qn[:, h * ND:(h + 1) * ND],
                                 kn[:, h * ND:(h + 1) * ND],
                                 (((1,), (1,)), ((), ())),
                                 preferred_element_type=jnp.float32)
        lg = lg + jax.lax.dot_general(qr_ref[h], kpe,
                                      (((1,), (1,)), ((), ())),
                                      preferred_element_type=jnp.float32)
        lg = jnp.where(mask, lg * scale, NEG)
        m = jnp.max(lg, axis=1, keepdims=True)
        e = jnp.exp(lg - m)
        p = (e / jnp.sum(e, axis=1, keepdims=True)).astype(jnp.bfloat16)
        ao = jnp.dot(p, v[:, h * VD:(h + 1) * VD],
                     preferred_element_type=jnp.float32).astype(jnp.bfloat16)
        acc = acc + jnp.dot(ao, wo_ref[h], preferred_element_type=jnp.float32)
    res2 = acc + res_ref[...]
    res2_ref[...] = res2
    h2_ref[...] = _rms(res2, plw_ref[...]).astype(jnp.bfloat16)


def _k5(qn, qr, kn, kpe, v, sc, th, wo_r, res, plw, g):
    W = (g + 1) * GR
    r0 = g * (GR // BT)
    import functools as _ft
    return pl.pallas_call(
        _ft.partial(_k5_body, g=g, W=W),
        grid=(GR // BT,),
        in_specs=[
            pl.BlockSpec((BT, H * ND), lambda i: (r0 + i, 0)),
            pl.BlockSpec((H, BT, RD), lambda i: (0, r0 + i, 0)),
            pl.BlockSpec((W, H * ND), lambda i: (0, 0)),
            pl.BlockSpec((W, RD), lambda i: (0, 0)),
            pl.BlockSpec((W, H * VD), lambda i: (0, 0)),
            pl.BlockSpec((BT, W), lambda i: (i, 0)),
            pl.BlockSpec((BT, 1), lambda i: (i, 0)),
            pl.BlockSpec((H, VD, D), lambda i: (0, 0, 0)),
            pl.BlockSpec((BT, D), lambda i: (r0 + i, 0)),
            pl.BlockSpec((1, D), lambda i: (0, 0)),
        ],
        out_specs=[
            pl.BlockSpec((BT, D), lambda i: (i, 0)),
            pl.BlockSpec((BT, D), lambda i: (i, 0)),
        ],
        out_shape=[
            jax.ShapeDtypeStruct((GR, D), jnp.float32),
            jax.ShapeDtypeStruct((GR, D), jnp.bfloat16),
        ],
    )(qn, qr, kn, kpe, v, sc, th, wo_r, res, plw.reshape(1, D))


def _k7_body(h2_ref, wg_ref, wu_ref, wd_ref, o_ref):
    @pl.when(pl.program_id(0) == 0)
    def _():
        o_ref[...] = jnp.zeros_like(o_ref)

    h2 = h2_ref[...]
    wg = wg_ref[...].astype(jnp.bfloat16)
    wu = wu_ref[...].astype(jnp.bfloat16)
    wd = wd_ref[...].astype(jnp.bfloat16)
    g = jnp.dot(h2, wg, preferred_element_type=jnp.float32)
    u = jnp.dot(h2, wu, preferred_element_type=jnp.float32)
    a = (g * jax.lax.logistic(g) * u).astype(jnp.bfloat16)
    o_ref[...] += jnp.dot(a, wd, preferred_element_type=jnp.float32)


def _k7(h2, wg, wu, wd):
    return pl.pallas_call(
        _k7_body,
        grid=(NBF,),
        in_specs=[
            pl.BlockSpec((T, D), lambda j: (0, 0)),
            pl.BlockSpec((D, BF), lambda j: (0, j)),
            pl.BlockSpec((D, BF), lambda j: (0, j)),
            pl.BlockSpec((BF, D), lambda j: (j, 0)),
        ],
        out_specs=pl.BlockSpec((T, D), lambda j: (0, 0)),
        out_shape=jax.ShapeDtypeStruct((T, D), jnp.float32),
        compiler_params=pltpu.CompilerParams(
            dimension_semantics=("arbitrary",)),
    )(h2, wg, wu, wd)


def kernel(positions, hidden_states, residual, input_ln_w, post_ln_w, W_qkv_a,
           q_a_ln_w, kv_a_ln_w, W_q_b, W_idx_k, idx_k_norm_w, idx_k_norm_b,
           W_idx_wts, W_idx_q_b, W_kv_b, W_o, W_gate, W_up, W_down):
    pos_col = positions.astype(jnp.float32).reshape(T, 1)
    wq = W_q_b.reshape(QL, H, ND + RD)
    wqn = wq[:, :, :ND].reshape(QL, H * ND).astype(jnp.bfloat16)
    wqr = jnp.transpose(wq[:, :, ND:], (1, 0, 2)).astype(jnp.bfloat16)
    wi = jnp.transpose(W_idx_q_b.reshape(QL, IH, ID), (1, 0, 2))
    wkv = W_kv_b.reshape(KVL, H, ND + VD)
    wkn = wkv[:, :, :ND].reshape(KVL, H * ND).astype(jnp.bfloat16)
    wv = wkv[:, :, ND:].reshape(KVL, H * VD).astype(jnp.bfloat16)
    wo_r = W_o.reshape(H, VD, D).astype(jnp.bfloat16)

    res, qc, kvc, kpe, ikp, wts = _k1(
        pos_col, hidden_states, residual, input_ln_w, W_qkv_a, q_a_ln_w,
        kv_a_ln_w, W_idx_k, idx_k_norm_w, idx_k_norm_b, W_idx_wts)
    qn, qr = _k2a(pos_col, qc, wqn, wqr)
    iqp = _k2i(pos_col, qc, wi)
    kn, v = _k2b(kvc, wkn, wv)
    res2_parts = []
    h2_parts = []
    for g in range(NG):
        sc_g, th_g = _k3(iqp, ikp, wts, g)
        r2_g, h2_g = _k5(qn, qr, kn, kpe, v, sc_g, th_g, wo_r, res,
                         post_ln_w, g)
        res2_parts.append(r2_g)
        h2_parts.append(h2_g)
    res2 = jnp.concatenate(res2_parts, axis=0)
    h2 = jnp.concatenate(h2_parts, axis=0)
    mlp_out = _k7(h2, W_gate, W_up, W_down)
    return (mlp_out, res2)
